# baseline (speedup 1.0000x reference)
import functools

import jax
import jax.numpy as jnp
from jax import lax
from jax.experimental import pallas as pl
from jax.experimental.pallas import tpu as pltpu
from jax.experimental.pallas import tpu_sc as plsc

VOCAB = 1000000
B = 4096
D = 64
W = 2 * D
NEG = 20
NEGP = 24
NC = 2
NS = 16
NW = NC * NS
BPW = B // NW
GRID = 8
BCHUNK = B // GRID


def _sc_body(v2_hbm, u2_hbm, chalf_hbm, thalf_hbm, ohalf_hbm,
             c2_out, t2_out, o2_out,
             chalf_v, thalf_v, ohalf_v, crows, trows, obuf,
             sem_c, sem_t, sem_o0, sem_o1):
    wid = lax.axis_index("s") * NC + lax.axis_index("c")
    base = wid * BPW

    pltpu.sync_copy(chalf_hbm.at[pl.ds(base, BPW)], chalf_v)
    pltpu.sync_copy(thalf_hbm.at[pl.ds(base, BPW)], thalf_v)
    pltpu.sync_copy(ohalf_hbm.at[pl.ds(wid * NEGP, NEGP)], ohalf_v)

    cdma = pltpu.async_copy(v2_hbm.at[chalf_v], crows, sem_c)
    tdma = pltpu.async_copy(u2_hbm.at[thalf_v], trows, sem_t)

    sems = (sem_o0, sem_o1)
    dmas = [
        pltpu.async_copy(u2_hbm.at[ohalf_v.at[0]], obuf.at[0], sems[0]),
        pltpu.async_copy(u2_hbm.at[ohalf_v.at[1]], obuf.at[1], sems[1]),
    ]
    for k in range(NEG):
        b = k % 2
        dmas[b].wait()
        pltpu.sync_copy(obuf.at[b], o2_out.at[k, pl.ds(base, BPW)])
        if k + 2 < NEG:
            dmas[b] = pltpu.async_copy(
                u2_hbm.at[ohalf_v.at[k + 2]], obuf.at[b], sems[b])

    cdma.wait()
    pltpu.sync_copy(crows, c2_out.at[pl.ds(base, BPW)])
    tdma.wait()
    pltpu.sync_copy(trows, t2_out.at[pl.ds(base, BPW)])


@jax.jit
def _sc_gather(v2, u2, chalf, thalf, ohalf2):
    mesh = plsc.VectorSubcoreMesh(
        core_axis_name="c", subcore_axis_name="s",
        num_cores=NC, num_subcores=NS)
    f = pl.kernel(
        _sc_body,
        out_type=(
            jax.ShapeDtypeStruct((B, W), jnp.float32),
            jax.ShapeDtypeStruct((B, W), jnp.float32),
            jax.ShapeDtypeStruct((NEG, B, W), jnp.float32),
        ),
        mesh=mesh,
        compiler_params=pltpu.CompilerParams(use_tc_tiling_on_sc=True),
        scratch_types=[
            pltpu.VMEM((BPW,), jnp.int32),
            pltpu.VMEM((BPW,), jnp.int32),
            pltpu.VMEM((NEGP, BPW), jnp.int32),
            pltpu.VMEM((BPW, W), jnp.float32),
            pltpu.VMEM((BPW, W), jnp.float32),
            pltpu.VMEM((2, BPW, W), jnp.float32),
            pltpu.SemaphoreType.DMA,
            pltpu.SemaphoreType.DMA,
            pltpu.SemaphoreType.DMA,
            pltpu.SemaphoreType.DMA,
        ],
    )
    return f(v2, u2, chalf, thalf, ohalf2)


def _log_sigmoid(x):
    return jnp.minimum(x, 0.0) - jnp.log1p(jnp.exp(-jnp.abs(x)))


def _loss_body(c2_ref, t2_ref, o2_ref, cpf_ref, tpf_ref, opf_ref, out_ref):
    i = pl.program_id(0)
    c2 = c2_ref[...]
    t2 = t2_ref[...]
    cpf = cpf_ref[...]
    tpf = tpf_ref[...]
    csel = c2[:, :D] * (1.0 - cpf) + c2[:, D:] * cpf
    tsel = t2[:, :D] * (1.0 - tpf) + t2[:, D:] * tpf
    o2 = o2_ref[...]
    opf = opf_ref[...][:, :, None]
    osel = o2[:, :, :D] * (1.0 - opf) + o2[:, :, D:] * opf
    usum = jnp.sum(osel, axis=0)
    p = jnp.sum(csel * tsel, axis=1)
    n = jnp.sum(csel * usum, axis=1)
    part = jnp.sum(_log_sigmoid(p)) + jnp.sum(_log_sigmoid(n))

    @pl.when(i == 0)
    def _():
        out_ref[...] = jnp.zeros((1, 1), jnp.float32)

    out_ref[...] += jnp.full((1, 1), -part / B, jnp.float32)


@jax.jit
def _loss(c2, t2, o2, cpf, tpf, opf):
    return pl.pallas_call(
        _loss_body,
        grid=(GRID,),
        in_specs=[
            pl.BlockSpec((BCHUNK, W), lambda i: (i, 0)),
            pl.BlockSpec((BCHUNK, W), lambda i: (i, 0)),
            pl.BlockSpec((NEG, BCHUNK, W), lambda i: (0, i, 0)),
            pl.BlockSpec((BCHUNK, 1), lambda i: (i, 0)),
            pl.BlockSpec((BCHUNK, 1), lambda i: (i, 0)),
            pl.BlockSpec((NEG, BCHUNK), lambda i: (0, i)),
        ],
        out_specs=pl.BlockSpec((1, 1), lambda i: (0, 0)),
        out_shape=jax.ShapeDtypeStruct((1, 1), jnp.float32),
    )(c2, t2, o2, cpf, tpf, opf)


def kernel(embedding_v, embedding_u, center_words, target_words, outer_words):
    cidx = center_words.reshape(B).astype(jnp.int32)
    tidx = target_words.reshape(B).astype(jnp.int32)
    oidx = outer_words.astype(jnp.int32)
    chalf = cidx // 2
    thalf = tidx // 2
    cpf = (cidx % 2).astype(jnp.float32).reshape(B, 1)
    tpf = (tidx % 2).astype(jnp.float32).reshape(B, 1)
    o3 = oidx.reshape(NW, BPW, NEG).transpose(0, 2, 1)
    ohalf2 = jnp.pad((o3 // 2), ((0, 0), (0, NEGP - NEG), (0, 0))
                     ).reshape(NW * NEGP, BPW)
    opf = (o3 % 2).astype(jnp.float32).transpose(1, 0, 2).reshape(NEG, B)
    v2 = embedding_v.reshape(VOCAB // 2, W)
    u2 = embedding_u.reshape(VOCAB // 2, W)
    c2, t2, o2 = _sc_gather(v2, u2, chalf, thalf, ohalf2)
    out = _loss(c2, t2, o2, cpf, tpf, opf)
    return out[0, 0]


# speedup vs baseline: 1.9400x; 1.9400x over previous
import functools

import jax
import jax.numpy as jnp
from jax import lax
from jax.experimental import pallas as pl
from jax.experimental.pallas import tpu as pltpu
from jax.experimental.pallas import tpu_sc as plsc

VOCAB = 1000000
B = 4096
D = 64
W = 2 * D
NEG = 20
NEGP = 24
NC = 2
NS = 16
NW = NC * NS
BPW = B // NW
GRID = 8
BCHUNK = B // GRID


def _sc_body(v2_hbm, u2_hbm, chalf_hbm, thalf_hbm, ohalf_hbm,
             c2_out, t2_out, o2_out,
             chalf_v, thalf_v, ohalf_v, crows, trows, obuf,
             sem_c, sem_t, sem_o0, sem_o1):
    wid = lax.axis_index("s") * NC + lax.axis_index("c")
    base = wid * BPW

    pltpu.sync_copy(chalf_hbm.at[pl.ds(base, BPW)], chalf_v)
    pltpu.sync_copy(thalf_hbm.at[pl.ds(base, BPW)], thalf_v)
    pltpu.sync_copy(ohalf_hbm.at[pl.ds(wid * NEGP, NEGP)], ohalf_v)

    cdma = pltpu.async_copy(v2_hbm.at[chalf_v], crows, sem_c)
    tdma = pltpu.async_copy(u2_hbm.at[thalf_v], trows, sem_t)

    sems = (sem_o0, sem_o1)
    dmas = [
        pltpu.async_copy(u2_hbm.at[ohalf_v.at[0]], obuf.at[0], sems[0]),
        pltpu.async_copy(u2_hbm.at[ohalf_v.at[1]], obuf.at[1], sems[1]),
    ]
    for k in range(NEG):
        b = k % 2
        dmas[b].wait()
        pltpu.sync_copy(obuf.at[b], o2_out.at[k, pl.ds(base, BPW)])
        if k + 2 < NEG:
            dmas[b] = pltpu.async_copy(
                u2_hbm.at[ohalf_v.at[k + 2]], obuf.at[b], sems[b])

    cdma.wait()
    pltpu.sync_copy(crows, c2_out.at[pl.ds(base, BPW)])
    tdma.wait()
    pltpu.sync_copy(trows, t2_out.at[pl.ds(base, BPW)])


@jax.jit
def _sc_gather(v2, u2, chalf, thalf, ohalf2):
    mesh = plsc.VectorSubcoreMesh(
        core_axis_name="c", subcore_axis_name="s",
        num_cores=NC, num_subcores=NS)
    f = pl.kernel(
        _sc_body,
        out_type=(
            jax.ShapeDtypeStruct((B, W), jnp.float32),
            jax.ShapeDtypeStruct((B, W), jnp.float32),
            jax.ShapeDtypeStruct((NEG, B, W), jnp.float32),
        ),
        mesh=mesh,
        compiler_params=pltpu.CompilerParams(use_tc_tiling_on_sc=True),
        scratch_types=[
            pltpu.VMEM((BPW,), jnp.int32),
            pltpu.VMEM((BPW,), jnp.int32),
            pltpu.VMEM((NEGP, BPW), jnp.int32),
            pltpu.VMEM((BPW, W), jnp.float32),
            pltpu.VMEM((BPW, W), jnp.float32),
            pltpu.VMEM((2, BPW, W), jnp.float32),
            pltpu.SemaphoreType.DMA,
            pltpu.SemaphoreType.DMA,
            pltpu.SemaphoreType.DMA,
            pltpu.SemaphoreType.DMA,
        ],
    )
    return f(v2, u2, chalf, thalf, ohalf2)


CCH = 8192
CH = CCH // 2
CGRID = -(-VOCAB // CCH)
VOUT = CGRID * CH


def _compact_body(xt_ref, out_ref):
    y = xt_ref[...].T
    out_ref[:, :D] = y[:CH, :]
    out_ref[:, D:] = y[CH:, :]


@jax.jit
def _compact(xt):
    return pl.pallas_call(
        _compact_body,
        grid=(CGRID,),
        in_specs=[pl.BlockSpec((D, CCH), lambda i: (0, i))],
        out_specs=pl.BlockSpec((CH, W), lambda i: (i, 0)),
        out_shape=jax.ShapeDtypeStruct((VOUT, W), jnp.float32),
    )(xt)


def _split_idx(idx):
    blk, off = idx // CCH, idx % CCH
    return blk * CH + off % CH, off // CH


def _log_sigmoid(x):
    return jnp.minimum(x, 0.0) - jnp.log1p(jnp.exp(-jnp.abs(x)))


def _loss_body(c2_ref, t2_ref, o2_ref, cpf_ref, tpf_ref, opf_ref, out_ref):
    i = pl.program_id(0)
    c2 = c2_ref[...]
    t2 = t2_ref[...]
    cpf = cpf_ref[...]
    tpf = tpf_ref[...]
    csel = c2[:, :D] * (1.0 - cpf) + c2[:, D:] * cpf
    tsel = t2[:, :D] * (1.0 - tpf) + t2[:, D:] * tpf
    o2 = o2_ref[...]
    opf = opf_ref[...][:, :, None]
    osel = o2[:, :, :D] * (1.0 - opf) + o2[:, :, D:] * opf
    usum = jnp.sum(osel, axis=0)
    p = jnp.sum(csel * tsel, axis=1)
    n = jnp.sum(csel * usum, axis=1)
    part = jnp.sum(_log_sigmoid(p)) + jnp.sum(_log_sigmoid(n))

    @pl.when(i == 0)
    def _():
        out_ref[...] = jnp.zeros((1, 1), jnp.float32)

    out_ref[...] += jnp.full((1, 1), -part / B, jnp.float32)


@jax.jit
def _loss(c2, t2, o2, cpf, tpf, opf):
    return pl.pallas_call(
        _loss_body,
        grid=(GRID,),
        in_specs=[
            pl.BlockSpec((BCHUNK, W), lambda i: (i, 0)),
            pl.BlockSpec((BCHUNK, W), lambda i: (i, 0)),
            pl.BlockSpec((NEG, BCHUNK, W), lambda i: (0, i, 0)),
            pl.BlockSpec((BCHUNK, 1), lambda i: (i, 0)),
            pl.BlockSpec((BCHUNK, 1), lambda i: (i, 0)),
            pl.BlockSpec((NEG, BCHUNK), lambda i: (0, i)),
        ],
        out_specs=pl.BlockSpec((1, 1), lambda i: (0, 0)),
        out_shape=jax.ShapeDtypeStruct((1, 1), jnp.float32),
    )(c2, t2, o2, cpf, tpf, opf)


def kernel(embedding_v, embedding_u, center_words, target_words, outer_words):
    cidx = center_words.reshape(B).astype(jnp.int32)
    tidx = target_words.reshape(B).astype(jnp.int32)
    oidx = outer_words.astype(jnp.int32)
    chalf, chf = _split_idx(cidx)
    thalf, thf = _split_idx(tidx)
    cpf = chf.astype(jnp.float32).reshape(B, 1)
    tpf = thf.astype(jnp.float32).reshape(B, 1)
    o3 = oidx.reshape(NW, BPW, NEG).transpose(0, 2, 1)
    oh3, of3 = _split_idx(o3)
    ohalf2 = jnp.pad(oh3, ((0, 0), (0, NEGP - NEG), (0, 0))
                     ).reshape(NW * NEGP, BPW)
    opf = of3.astype(jnp.float32).transpose(1, 0, 2).reshape(NEG, B)
    v2 = _compact(embedding_v.T)
    u2 = _compact(embedding_u.T)
    c2, t2, o2 = _sc_gather(v2, u2, chalf, thalf, ohalf2)
    out = _loss(c2, t2, o2, cpf, tpf, opf)
    return out[0, 0]


# speedup vs baseline: 2.1618x; 1.1143x over previous
import functools

import jax
import jax.numpy as jnp
from jax import lax
from jax.experimental import pallas as pl
from jax.experimental.pallas import tpu as pltpu
from jax.experimental.pallas import tpu_sc as plsc

VOCAB = 1000000
B = 4096
D = 64
W = 2 * D
NEG = 20
NEGP = 24
NC = 2
NS = 16
NW = NC * NS
BPW = B // NW
GRID = 8
BCHUNK = B // GRID


def _sc_body(v2_hbm, u2_hbm, chalf_hbm, thalf_hbm, ohalf_hbm,
             c2_out, t2_out, o2_out,
             chalf_v, thalf_v, ohalf_v, crows, trows, obuf,
             sem_c, sem_t, sem_o0, sem_o1):
    wid = lax.axis_index("s") * NC + lax.axis_index("c")
    base = wid * BPW

    pltpu.sync_copy(chalf_hbm.at[pl.ds(base, BPW)], chalf_v)
    pltpu.sync_copy(thalf_hbm.at[pl.ds(base, BPW)], thalf_v)
    pltpu.sync_copy(ohalf_hbm.at[pl.ds(wid * NEGP, NEGP)], ohalf_v)

    cdma = pltpu.async_copy(v2_hbm.at[chalf_v], crows, sem_c)
    tdma = pltpu.async_copy(u2_hbm.at[thalf_v], trows, sem_t)

    sems = (sem_o0, sem_o1)
    dmas = [
        pltpu.async_copy(u2_hbm.at[ohalf_v.at[0]], obuf.at[0], sems[0]),
        pltpu.async_copy(u2_hbm.at[ohalf_v.at[1]], obuf.at[1], sems[1]),
    ]
    for k in range(NEG):
        b = k % 2
        dmas[b].wait()
        pltpu.sync_copy(obuf.at[b], o2_out.at[k, pl.ds(base, BPW)])
        if k + 2 < NEG:
            dmas[b] = pltpu.async_copy(
                u2_hbm.at[ohalf_v.at[k + 2]], obuf.at[b], sems[b])

    cdma.wait()
    pltpu.sync_copy(crows, c2_out.at[pl.ds(base, BPW)])
    tdma.wait()
    pltpu.sync_copy(trows, t2_out.at[pl.ds(base, BPW)])


@jax.jit
def _sc_gather(v2, u2, chalf, thalf, ohalf2):
    mesh = plsc.VectorSubcoreMesh(
        core_axis_name="c", subcore_axis_name="s",
        num_cores=NC, num_subcores=NS)
    f = pl.kernel(
        _sc_body,
        out_type=(
            jax.ShapeDtypeStruct((B, W), jnp.int32),
            jax.ShapeDtypeStruct((B, W), jnp.int32),
            jax.ShapeDtypeStruct((NEG, B, W), jnp.int32),
        ),
        mesh=mesh,
        compiler_params=pltpu.CompilerParams(use_tc_tiling_on_sc=True),
        scratch_types=[
            pltpu.VMEM((BPW,), jnp.int32),
            pltpu.VMEM((BPW,), jnp.int32),
            pltpu.VMEM((NEGP, BPW), jnp.int32),
            pltpu.VMEM((BPW, W), jnp.int32),
            pltpu.VMEM((BPW, W), jnp.int32),
            pltpu.VMEM((2, BPW, W), jnp.int32),
            pltpu.SemaphoreType.DMA,
            pltpu.SemaphoreType.DMA,
            pltpu.SemaphoreType.DMA,
            pltpu.SemaphoreType.DMA,
        ],
    )
    return f(v2, u2, chalf, thalf, ohalf2)


CCH = 8192
CGRID = -(-VOCAB // CCH)
VOUT = CGRID * (CCH // 4)


MHI = -65536
Q = CCH // 4


def _compact_body(xt_ref, out_ref):
    y = xt_ref[...].T
    z = jax.lax.bitcast_convert_type(y, jnp.int32)
    lw = (z[:Q] & MHI) | jax.lax.shift_right_logical(z[Q:2 * Q], 16)
    rw = (z[2 * Q:3 * Q] & MHI) | jax.lax.shift_right_logical(z[3 * Q:], 16)
    out_ref[...] = jnp.concatenate([lw, rw], axis=1)


@jax.jit
def _compact(xt):
    return pl.pallas_call(
        _compact_body,
        grid=(CGRID,),
        in_specs=[pl.BlockSpec((D, CCH), lambda i: (0, i))],
        out_specs=pl.BlockSpec((Q, W), lambda i: (i, 0)),
        out_shape=jax.ShapeDtypeStruct((VOUT, W), jnp.int32),
    )(xt)


def _split_idx(idx):
    blk, off = idx // CCH, idx % CCH
    q = off // Q
    return blk * Q + off % Q, q // 2, q % 2


def _log_sigmoid(x):
    return jnp.minimum(x, 0.0) - jnp.log1p(jnp.exp(-jnp.abs(x)))


def _unpack(words, win, hl):
    w = jnp.where(win > 0, words[..., D:], words[..., :D])
    w = jnp.where(hl > 0, jax.lax.shift_left(w, 16), w & MHI)
    return jax.lax.bitcast_convert_type(w, jnp.float32)


def _loss_body(c2_ref, t2_ref, o2_ref, cs_ref, ts_ref, ow_ref, oh_ref,
               out_ref):
    i = pl.program_id(0)
    cs = cs_ref[...]
    ts = ts_ref[...]
    csel = _unpack(c2_ref[...], cs[:, :1], cs[:, 1:])
    tsel = _unpack(t2_ref[...], ts[:, :1], ts[:, 1:])
    osel = _unpack(o2_ref[...], ow_ref[...][:, :, None], oh_ref[...][:, :, None])
    usum = jnp.sum(osel, axis=0)
    p = jnp.sum(csel * tsel, axis=1)
    n = jnp.sum(csel * usum, axis=1)
    part = jnp.sum(_log_sigmoid(p)) + jnp.sum(_log_sigmoid(n))

    @pl.when(i == 0)
    def _():
        out_ref[...] = jnp.zeros((1, 1), jnp.float32)

    out_ref[...] += jnp.full((1, 1), -part / B, jnp.float32)


@jax.jit
def _loss(c2, t2, o2, csel, tsel, owin, ohl):
    return pl.pallas_call(
        _loss_body,
        grid=(GRID,),
        in_specs=[
            pl.BlockSpec((BCHUNK, W), lambda i: (i, 0)),
            pl.BlockSpec((BCHUNK, W), lambda i: (i, 0)),
            pl.BlockSpec((NEG, BCHUNK, W), lambda i: (0, i, 0)),
            pl.BlockSpec((BCHUNK, 2), lambda i: (i, 0)),
            pl.BlockSpec((BCHUNK, 2), lambda i: (i, 0)),
            pl.BlockSpec((NEG, BCHUNK), lambda i: (0, i)),
            pl.BlockSpec((NEG, BCHUNK), lambda i: (0, i)),
        ],
        out_specs=pl.BlockSpec((1, 1), lambda i: (0, 0)),
        out_shape=jax.ShapeDtypeStruct((1, 1), jnp.float32),
    )(c2, t2, o2, csel, tsel, owin, ohl)


def kernel(embedding_v, embedding_u, center_words, target_words, outer_words):
    cidx = center_words.reshape(B).astype(jnp.int32)
    tidx = target_words.reshape(B).astype(jnp.int32)
    oidx = outer_words.astype(jnp.int32)
    crow, cwin, chl = _split_idx(cidx)
    trow, twin, thl = _split_idx(tidx)
    csel = jnp.stack([cwin, chl], axis=1)
    tsel = jnp.stack([twin, thl], axis=1)
    o3 = oidx.reshape(NW, BPW, NEG).transpose(0, 2, 1)
    orow3, owin3, ohl3 = _split_idx(o3)
    orow2 = jnp.pad(orow3, ((0, 0), (0, NEGP - NEG), (0, 0))
                    ).reshape(NW * NEGP, BPW)
    owin = owin3.transpose(1, 0, 2).reshape(NEG, B)
    ohl = ohl3.transpose(1, 0, 2).reshape(NEG, B)
    v2 = _compact(embedding_v.T)
    u2 = _compact(embedding_u.T)
    c2, t2, o2 = _sc_gather(v2, u2, crow, trow, orow2)
    out = _loss(c2, t2, o2, csel, tsel, owin, ohl)
    return out[0, 0]


# speedup vs baseline: 2.4839x; 1.1490x over previous
import functools

import jax
import jax.numpy as jnp
from jax import lax
from jax.experimental import pallas as pl
from jax.experimental.pallas import tpu as pltpu
from jax.experimental.pallas import tpu_sc as plsc

VOCAB = 1000000
B = 4096
D = 64
W = 2 * D
NEG = 20
NEGP = 24
NC = 2
NS = 16
NW = NC * NS
BPW = B // NW
GRID = 8
BCHUNK = B // GRID


def _mesh():
    return plsc.VectorSubcoreMesh(
        core_axis_name="c", subcore_axis_name="s",
        num_cores=NC, num_subcores=NS)


def _wid_base():
    wid = lax.axis_index("s") * NC + lax.axis_index("c")
    return wid, wid * BPW


def _sc_body_u(u2_hbm, thalf_hbm, ohalf_hbm, t2_out, o2_out,
               thalf_v, ohalf_v, trows, obuf, sem_t, sem_o0, sem_o1):
    wid, base = _wid_base()
    pltpu.sync_copy(thalf_hbm.at[pl.ds(base, BPW)], thalf_v)
    pltpu.sync_copy(ohalf_hbm.at[pl.ds(wid * NEGP, NEGP)], ohalf_v)

    tdma = pltpu.async_copy(u2_hbm.at[thalf_v], trows, sem_t)

    sems = (sem_o0, sem_o1)
    dmas = [
        pltpu.async_copy(u2_hbm.at[ohalf_v.at[0]], obuf.at[0], sems[0]),
        pltpu.async_copy(u2_hbm.at[ohalf_v.at[1]], obuf.at[1], sems[1]),
    ]
    for k in range(NEG):
        b = k % 2
        dmas[b].wait()
        pltpu.sync_copy(obuf.at[b], o2_out.at[k, pl.ds(base, BPW)])
        if k + 2 < NEG:
            dmas[b] = pltpu.async_copy(
                u2_hbm.at[ohalf_v.at[k + 2]], obuf.at[b], sems[b])

    tdma.wait()
    pltpu.sync_copy(trows, t2_out.at[pl.ds(base, BPW)])


def _sc_body_v(v2_hbm, chalf_hbm, c2_out, chalf_v, crows, sem_c):
    _, base = _wid_base()
    pltpu.sync_copy(chalf_hbm.at[pl.ds(base, BPW)], chalf_v)
    pltpu.async_copy(v2_hbm.at[chalf_v], crows, sem_c).wait()
    pltpu.sync_copy(crows, c2_out.at[pl.ds(base, BPW)])


@jax.jit
def _sc_gather_u(u2, thalf, ohalf2):
    f = pl.kernel(
        _sc_body_u,
        out_type=(
            jax.ShapeDtypeStruct((B, W), jnp.int32),
            jax.ShapeDtypeStruct((NEG, B, W), jnp.int32),
        ),
        mesh=_mesh(),
        compiler_params=pltpu.CompilerParams(use_tc_tiling_on_sc=True),
        scratch_types=[
            pltpu.VMEM((BPW,), jnp.int32),
            pltpu.VMEM((NEGP, BPW), jnp.int32),
            pltpu.VMEM((BPW, W), jnp.int32),
            pltpu.VMEM((2, BPW, W), jnp.int32),
            pltpu.SemaphoreType.DMA,
            pltpu.SemaphoreType.DMA,
            pltpu.SemaphoreType.DMA,
        ],
    )
    return f(u2, thalf, ohalf2)


@jax.jit
def _sc_gather_v(v2, chalf):
    f = pl.kernel(
        _sc_body_v,
        out_type=jax.ShapeDtypeStruct((B, W), jnp.int32),
        mesh=_mesh(),
        compiler_params=pltpu.CompilerParams(use_tc_tiling_on_sc=True),
        scratch_types=[
            pltpu.VMEM((BPW,), jnp.int32),
            pltpu.VMEM((BPW, W), jnp.int32),
            pltpu.SemaphoreType.DMA,
        ],
    )
    return f(v2, chalf)


CCH = 16384
CGRID = -(-VOCAB // CCH)
VOUT = CGRID * (CCH // 4)


MHI = -65536
Q = CCH // 4


def _compact_body(xt_ref, out_ref):
    y = xt_ref[...].T
    z = jax.lax.bitcast_convert_type(y, jnp.int32)
    lw = (z[:Q] & MHI) | jax.lax.shift_right_logical(z[Q:2 * Q], 16)
    rw = (z[2 * Q:3 * Q] & MHI) | jax.lax.shift_right_logical(z[3 * Q:], 16)
    out_ref[...] = jnp.concatenate([lw, rw], axis=1)


@jax.jit
def _compact(xt):
    return pl.pallas_call(
        _compact_body,
        grid=(CGRID,),
        in_specs=[pl.BlockSpec((D, CCH), lambda i: (0, i))],
        out_specs=pl.BlockSpec((Q, W), lambda i: (i, 0)),
        out_shape=jax.ShapeDtypeStruct((VOUT, W), jnp.int32),
    )(xt)


def _split_idx(idx):
    blk, off = idx // CCH, idx % CCH
    q = off // Q
    return blk * Q + off % Q, q // 2, q % 2


def _log_sigmoid(x):
    return jnp.minimum(x, 0.0) - jnp.log1p(jnp.exp(-jnp.abs(x)))


def _unpack(words, win, hl):
    w = jnp.where(win > 0, words[..., D:], words[..., :D])
    w = jnp.where(hl > 0, jax.lax.shift_left(w, 16), w & MHI)
    return jax.lax.bitcast_convert_type(w, jnp.float32)


def _loss_body(c2_ref, t2_ref, o2_ref, cs_ref, ts_ref, ow_ref, oh_ref,
               out_ref):
    i = pl.program_id(0)
    cs = cs_ref[...]
    ts = ts_ref[...]
    csel = _unpack(c2_ref[...], cs[:, :1], cs[:, 1:])
    tsel = _unpack(t2_ref[...], ts[:, :1], ts[:, 1:])
    osel = _unpack(o2_ref[...], ow_ref[...][:, :, None], oh_ref[...][:, :, None])
    usum = jnp.sum(osel, axis=0)
    p = jnp.sum(csel * tsel, axis=1)
    n = jnp.sum(csel * usum, axis=1)
    part = jnp.sum(_log_sigmoid(p)) + jnp.sum(_log_sigmoid(n))

    @pl.when(i == 0)
    def _():
        out_ref[...] = jnp.zeros((1, 1), jnp.float32)

    out_ref[...] += jnp.full((1, 1), -part / B, jnp.float32)


@jax.jit
def _loss(c2, t2, o2, csel, tsel, owin, ohl):
    return pl.pallas_call(
        _loss_body,
        grid=(GRID,),
        in_specs=[
            pl.BlockSpec((BCHUNK, W), lambda i: (i, 0)),
            pl.BlockSpec((BCHUNK, W), lambda i: (i, 0)),
            pl.BlockSpec((NEG, BCHUNK, W), lambda i: (0, i, 0)),
            pl.BlockSpec((BCHUNK, 2), lambda i: (i, 0)),
            pl.BlockSpec((BCHUNK, 2), lambda i: (i, 0)),
            pl.BlockSpec((NEG, BCHUNK), lambda i: (0, i)),
            pl.BlockSpec((NEG, BCHUNK), lambda i: (0, i)),
        ],
        out_specs=pl.BlockSpec((1, 1), lambda i: (0, 0)),
        out_shape=jax.ShapeDtypeStruct((1, 1), jnp.float32),
    )(c2, t2, o2, csel, tsel, owin, ohl)


def kernel(embedding_v, embedding_u, center_words, target_words, outer_words):
    cidx = center_words.reshape(B).astype(jnp.int32)
    tidx = target_words.reshape(B).astype(jnp.int32)
    oidx = outer_words.astype(jnp.int32)
    crow, cwin, chl = _split_idx(cidx)
    trow, twin, thl = _split_idx(tidx)
    csel = jnp.stack([cwin, chl], axis=1)
    tsel = jnp.stack([twin, thl], axis=1)
    o3 = oidx.reshape(NW, BPW, NEG).transpose(0, 2, 1)
    orow3, owin3, ohl3 = _split_idx(o3)
    orow2 = jnp.pad(orow3, ((0, 0), (0, NEGP - NEG), (0, 0))
                    ).reshape(NW * NEGP, BPW)
    owin = owin3.transpose(1, 0, 2).reshape(NEG, B)
    ohl = ohl3.transpose(1, 0, 2).reshape(NEG, B)
    u2 = _compact(embedding_u.T)
    t2, o2 = _sc_gather_u(u2, trow, orow2)
    v2 = _compact(embedding_v.T)
    c2 = _sc_gather_v(v2, crow)
    out = _loss(c2, t2, o2, csel, tsel, owin, ohl)
    return out[0, 0]
